# baseline (speedup 1.0000x reference)
import functools

import jax
import jax.numpy as jnp
from jax import lax
from jax.experimental import pallas as pl
from jax.experimental.pallas import tpu as pltpu
from jax.experimental.pallas import tpu_sc as plsc

_K = 8192
_D = 32
_B = 64
_T = 1024
_N = _B * _T

_info = plsc.get_sparse_core_info()
_NC, _NS, _L = _info.num_cores, _info.num_subcores, _info.num_lanes
_NW = _NC * _NS
_N_PER_W = _N // _NW
_PH = 128 // _D
_VR_PER_W = _N_PER_W // _PH


@functools.partial(
    pl.kernel,
    mesh=plsc.VectorSubcoreMesh(core_axis_name="c", subcore_axis_name="s"),
    out_type=jax.ShapeDtypeStruct((_N * _D // 128, 128), jnp.float32),
    scratch_types=[
        pltpu.VMEM((_N_PER_W,), jnp.int32),
        pltpu.VMEM((_VR_PER_W, _D), jnp.float32),
        pltpu.VMEM((_VR_PER_W, _D), jnp.float32),
        pltpu.VMEM((_VR_PER_W, _D), jnp.float32),
        pltpu.VMEM((_VR_PER_W, _D), jnp.float32),
        pltpu.SemaphoreType.DMA,
        pltpu.SemaphoreType.DMA,
        pltpu.SemaphoreType.DMA,
        pltpu.SemaphoreType.DMA,
    ],
    compiler_params=pltpu.CompilerParams(use_tc_tiling_on_sc=False),
)
def _gather_rows(idx_hbm, table_hbm, out_hbm, idx_v, r0, r1, r2, r3,
                 s0, s1, s2, s3):
    wid = lax.axis_index("s") * _NC + lax.axis_index("c")
    pltpu.sync_copy(idx_hbm.at[pl.ds(wid * _N_PER_W, _N_PER_W)], idx_v)
    rbufs = (r0, r1, r2, r3)
    sems = (s0, s1, s2, s3)
    copies = [
        pltpu.async_copy(
            table_hbm.at[idx_v.at[pl.ds(j * _VR_PER_W, _VR_PER_W)]],
            rbufs[j],
            sems[j],
        )
        for j in range(_PH)
    ]
    vbase = wid * _VR_PER_W
    for j in range(_PH):
        copies[j].wait()
        pltpu.sync_copy(
            rbufs[j],
            out_hbm.at[pl.ds(vbase, _VR_PER_W), pl.ds(j * _D, _D)],
        )


@jax.jit
def kernel(embed_id, weight):
    ids4 = (
        embed_id.reshape(_NW, _VR_PER_W, _PH)
        .transpose(0, 2, 1)
        .reshape(_N)
    )
    out = _gather_rows(ids4, weight)
    return out.reshape(_B, _T, _D)


# speedup vs baseline: 1.2521x; 1.2521x over previous
import functools

import jax
import jax.numpy as jnp
from jax import lax
from jax.experimental import pallas as pl
from jax.experimental.pallas import tpu as pltpu
from jax.experimental.pallas import tpu_sc as plsc

_K = 8192
_D = 32
_B = 64
_T = 1024
_N = _B * _T

_info = plsc.get_sparse_core_info()
_NC, _NS = _info.num_cores, _info.num_subcores
_NW = _NC * _NS
_N_PER_W = _N // _NW


@functools.partial(
    pl.kernel,
    mesh=plsc.VectorSubcoreMesh(core_axis_name="c", subcore_axis_name="s"),
    out_type=jax.ShapeDtypeStruct((_N, _D), jnp.float32),
    scratch_types=[
        pltpu.VMEM((_N_PER_W,), jnp.int32),
        pltpu.VMEM((_N_PER_W, _D), jnp.float32),
        pltpu.SemaphoreType.DMA,
    ],
    compiler_params=pltpu.CompilerParams(use_tc_tiling_on_sc=False),
)
def _gather_rows(idx_hbm, table_hbm, out_hbm, idx_v, rows_v, sem):
    wid = lax.axis_index("s") * _NC + lax.axis_index("c")
    base = wid * _N_PER_W
    pltpu.sync_copy(idx_hbm.at[pl.ds(base, _N_PER_W)], idx_v)
    pltpu.async_copy(table_hbm.at[idx_v], rows_v, sem).wait()
    pltpu.sync_copy(rows_v, out_hbm.at[pl.ds(base, _N_PER_W)])


@jax.jit
def kernel(embed_id, weight):
    flat_ids = embed_id.reshape(_N)
    out = _gather_rows(flat_ids, weight)
    return out.reshape(_B, _T, _D)
